# Initial kernel scaffold; baseline (speedup 1.0000x reference)
import jax
import jax.numpy as jnp
from jax.experimental import pallas as pl


def kernel(x, y, memory):
    raise NotImplementedError("write your pallas kernel here")


# speedup vs baseline: 1.1597x; 1.1597x over previous
import functools

import jax
import jax.numpy as jnp
from jax import lax
from jax.experimental import pallas as pl
from jax.experimental.pallas import tpu as pltpu
from jax.experimental.pallas import tpu_sc as plsc

N_DATA = 1000000
N_DIM = 64
K = 2048
T = 0.07
M = 0.5
BSZ = 1024

NC = 2
NS = 16
NW = NC * NS
B_PER_W = BSZ // NW
ROWS_PER_W = N_DATA // NW
CHUNK = 512
N_CHUNK = K // CHUNK
INV_T = 1.0 / T

def _neg_idx():
    idx = jax.random.randint(
        jax.random.key(42), (BSZ, K + 1), 0, N_DATA, dtype=jnp.int32)
    return idx[:, 1:].reshape(BSZ, K // 128, 128)


def _rsqrt16(n):
    i = plsc.bitcast(n, jnp.int32)
    i = jnp.int32(0x5F3759DF) - (i >> 1)
    s = plsc.bitcast(i, jnp.float32)
    for _ in range(4):
        s = s * (1.5 - 0.5 * n * s * s)
    return s


def _wid():
    return lax.axis_index("s") * NC + lax.axis_index("c")


def _sc_main_body(x_hbm, y_hbm, mem_hbm, idx_hbm,
                  lrest_hbm, l0_hbm, upd_hbm,
                  xv, idxv, rows_a, rows_b, logv,
                  yv, xtv, wposv, updv, l0v,
                  sem_a, sem_b, sem_u):
    wid = _wid()
    iota = lax.iota(jnp.int32, 16)
    iota64 = iota * N_DIM
    base = wid * B_PER_W

    pltpu.sync_copy(y_hbm.at[pl.ds(base, B_PER_W)], yv)
    pltpu.async_copy(mem_hbm.at[yv], wposv, sem_u).wait()
    pltpu.sync_copy(x_hbm.at[pl.ds(base, B_PER_W)], xtv)
    rinvs = []
    for g in range(B_PER_W // 16):
        r16 = iota + g * 16
        accs = [jnp.zeros((16,), jnp.float32) for _ in range(4)]
        nrms = [jnp.zeros((16,), jnp.float32) for _ in range(4)]
        for d in range(N_DIM):
            col = jnp.full((16,), d, jnp.int32)
            wv = plsc.load_gather(wposv, [r16, col])
            xw = plsc.load_gather(xtv, [r16, col])
            accs[d % 4] = accs[d % 4] + wv * xw
            u = (wv + xw) * 0.5
            nrms[d % 4] = nrms[d % 4] + u * u
        dotv = (accs[0] + accs[1]) + (accs[2] + accs[3])
        nrm = (nrms[0] + nrms[1]) + (nrms[2] + nrms[3])
        l0v[pl.ds(g * 16, 16)] = dotv * INV_T
        rinvs.append(_rsqrt16(jnp.maximum(nrm, 1e-24)))

    for j in range(B_PER_W):
        rs = rinvs[j // 16][j % 16] * 0.5
        for q in range(N_DIM // 16):
            sl = pl.ds(q * 16, 16)
            updv[j, sl] = (wposv[j, sl] + xtv[j, sl]) * rs

    pltpu.sync_copy(updv, upd_hbm.at[pl.ds(base, B_PER_W)])
    pltpu.sync_copy(l0v, l0_hbm.at[pl.ds(base, B_PER_W)])

    def fire(c, rows_ref, sem):
        ds_ = []
        for q in range(CHUNK // 128):
            ds_.append(pltpu.async_copy(
                mem_hbm.at[idxv.at[c * (CHUNK // 128) + q]],
                rows_ref.at[pl.ds(q * 128, 128)], sem))
        return ds_

    def drain(ds_):
        for d in ds_:
            d.wait()

    def compute(c, rows_ref, xs):
        def body(i, _):
            r16 = iota + i * 16
            accs = [jnp.zeros((16,), jnp.float32) for _ in range(4)]
            for d in range(N_DIM):
                col = jnp.full((16,), d, jnp.int32)
                v = plsc.load_gather(rows_ref, [r16, col])
                accs[d % 4] = accs[d % 4] + v * xs[d]
            acc = (accs[0] + accs[1]) + (accs[2] + accs[3])
            logv[pl.ds(c * CHUNK + i * 16, 16)] = acc * INV_T
            return 0
        lax.fori_loop(0, CHUNK // 16, body, 0)

    def per_b(j, _):
        b = base + j
        pltpu.sync_copy(x_hbm.at[b], xv)
        pltpu.sync_copy(idx_hbm.at[b], idxv)
        xs = []
        for q in range(N_DIM // 16):
            xq = xv[pl.ds(q * 16, 16)]
            xs.extend(xq[l] for l in range(16))
        pend = fire(0, rows_a, sem_a)
        for c in range(N_CHUNK):
            cur_rows = rows_a if c % 2 == 0 else rows_b
            if c < N_CHUNK - 1:
                nxt = fire(c + 1, rows_b if c % 2 == 0 else rows_a,
                           sem_b if c % 2 == 0 else sem_a)
            drain(pend)
            compute(c, cur_rows, xs)
            if c < N_CHUNK - 1:
                pend = nxt
        pltpu.sync_copy(logv, lrest_hbm.at[b])
        return 0

    lax.fori_loop(0, B_PER_W, per_b, 0)


def _sc_scatter_body(y_hbm, mem_hbm, upd_hbm, newm_hbm, yv, trow):
    wid = _wid()
    start = wid * ROWS_PER_W
    pltpu.sync_copy(mem_hbm.at[pl.ds(start, ROWS_PER_W)],
                    newm_hbm.at[pl.ds(start, ROWS_PER_W)])
    pltpu.sync_copy(y_hbm, yv)

    def body(g, _):
        yvec = yv[pl.ds(g * 16, 16)]
        for l in range(16):
            yj = yvec[l]

            @pl.when(jnp.logical_and(yj >= start, yj < start + ROWS_PER_W))
            def _(yj=yj, l=l):
                pltpu.sync_copy(upd_hbm.at[pl.ds(g * 16 + l, 1)], trow)
                pltpu.sync_copy(trow, newm_hbm.at[pl.ds(yj, 1)])
        return 0

    lax.fori_loop(0, BSZ // 16, body, 0)


_MESH = plsc.VectorSubcoreMesh(core_axis_name="c", subcore_axis_name="s")

_sc_main = functools.partial(
    pl.kernel,
    mesh=_MESH,
    compiler_params=pltpu.CompilerParams(
        needs_layout_passes=False, use_tc_tiling_on_sc=False),
    out_type=[
        jax.ShapeDtypeStruct((BSZ, K), jnp.float32),
        jax.ShapeDtypeStruct((BSZ,), jnp.float32),
        jax.ShapeDtypeStruct((BSZ, N_DIM), jnp.float32),
    ],
    scratch_types=[
        pltpu.VMEM((N_DIM,), jnp.float32),
        pltpu.VMEM((K // 128, 128), jnp.int32),
        pltpu.VMEM((CHUNK, N_DIM), jnp.float32),
        pltpu.VMEM((CHUNK, N_DIM), jnp.float32),
        pltpu.VMEM((K,), jnp.float32),
        pltpu.VMEM((B_PER_W,), jnp.int32),
        pltpu.VMEM((B_PER_W, N_DIM), jnp.float32),
        pltpu.VMEM((B_PER_W, N_DIM), jnp.float32),
        pltpu.VMEM((B_PER_W, N_DIM), jnp.float32),
        pltpu.VMEM((B_PER_W,), jnp.float32),
        pltpu.SemaphoreType.DMA,
        pltpu.SemaphoreType.DMA,
        pltpu.SemaphoreType.DMA,
    ],
)(_sc_main_body)

_sc_scatter = functools.partial(
    pl.kernel,
    mesh=_MESH,
    compiler_params=pltpu.CompilerParams(
        needs_layout_passes=False, use_tc_tiling_on_sc=False),
    out_type=[jax.ShapeDtypeStruct((N_DATA, N_DIM), jnp.float32)],
    scratch_types=[
        pltpu.VMEM((BSZ,), jnp.int32),
        pltpu.VMEM((1, N_DIM), jnp.float32),
    ],
)(_sc_scatter_body)


def kernel(x, y, memory):
    idxc = _neg_idx()
    lrest, l0, upd = _sc_main(x, y, memory, idxc)
    (new_memory,) = _sc_scatter(y, memory, upd)
    logits = jnp.concatenate([l0[:, None], lrest], axis=1)
    labels = jnp.zeros((BSZ,), jnp.int32)
    return logits, labels, new_memory


# speedup vs baseline: 5.9331x; 5.1162x over previous
import functools

import jax
import jax.numpy as jnp
from jax import lax
from jax.experimental import pallas as pl
from jax.experimental.pallas import tpu as pltpu
from jax.experimental.pallas import tpu_sc as plsc

N_DATA = 1000000
N_DIM = 64
K = 2048
T = 0.07
M = 0.5
BSZ = 1024

NC = 2
NS = 16
NW = NC * NS
B_PER_W = BSZ // NW
ROWS_PER_W = N_DATA // NW
CHUNK = 512
N_CHUNK = K // CHUNK
CP_CHUNK = 625
N_CP = ROWS_PER_W // CP_CHUNK
INV_T = 1.0 / T

def _neg_idx():
    idx = jax.random.randint(
        jax.random.key(42), (BSZ, K + 1), 0, N_DATA, dtype=jnp.int32)
    return idx[:, 1:].reshape(BSZ, K // 128, 128)


def _rsqrt16(n):
    i = plsc.bitcast(n, jnp.int32)
    i = jnp.int32(0x5F3759DF) - (i >> 1)
    s = plsc.bitcast(i, jnp.float32)
    for _ in range(4):
        s = s * (1.5 - 0.5 * n * s * s)
    return s


def _wid():
    return lax.axis_index("s") * NC + lax.axis_index("c")


def _sc_main_body(x_hbm, y_hbm, mem_hbm, idx_hbm,
                  lrest_hbm, l0_hbm, upd_hbm,
                  xv, idxv, rows_a, rows_b, xrott, logv,
                  yv, xtv, wposv, updv, l0v,
                  sem_a, sem_b, sem_u):
    wid = _wid()
    iota = lax.iota(jnp.int32, 16)
    iota64 = iota * N_DIM
    base = wid * B_PER_W

    pltpu.sync_copy(y_hbm.at[pl.ds(base, B_PER_W)], yv)
    pltpu.async_copy(mem_hbm.at[yv], wposv, sem_u).wait()
    pltpu.sync_copy(x_hbm.at[pl.ds(base, B_PER_W)], xtv)
    rinvs = []
    for g in range(B_PER_W // 16):
        r16 = iota + g * 16
        accs = [jnp.zeros((16,), jnp.float32) for _ in range(4)]
        nrms = [jnp.zeros((16,), jnp.float32) for _ in range(4)]
        for d in range(N_DIM):
            col = jnp.full((16,), d, jnp.int32)
            wv = plsc.load_gather(wposv, [r16, col])
            xw = plsc.load_gather(xtv, [r16, col])
            accs[d % 4] = accs[d % 4] + wv * xw
            u = (wv + xw) * 0.5
            nrms[d % 4] = nrms[d % 4] + u * u
        dotv = (accs[0] + accs[1]) + (accs[2] + accs[3])
        nrm = (nrms[0] + nrms[1]) + (nrms[2] + nrms[3])
        l0v[pl.ds(g * 16, 16)] = dotv * INV_T
        rinvs.append(_rsqrt16(jnp.maximum(nrm, 1e-24)))

    for j in range(B_PER_W):
        rs = rinvs[j // 16][j % 16] * 0.5
        for q in range(N_DIM // 16):
            sl = pl.ds(q * 16, 16)
            updv[j, sl] = (wposv[j, sl] + xtv[j, sl]) * rs

    pltpu.sync_copy(updv, upd_hbm.at[pl.ds(base, B_PER_W)])
    pltpu.sync_copy(l0v, l0_hbm.at[pl.ds(base, B_PER_W)])

    def fire(c, rows_ref, sem):
        ds_ = []
        for q in range(CHUNK // 128):
            ds_.append(pltpu.async_copy(
                mem_hbm.at[idxv.at[c * (CHUNK // 128) + q]],
                rows_ref.at[pl.ds(q * 128, 128)], sem))
        return ds_

    def drain(ds_):
        for d in ds_:
            d.wait()

    def compute(c, rows_ref):
        def body(i, _):
            r16 = iota + i * 16
            rot = iota
            accs = [jnp.zeros((16,), jnp.float32) for _ in range(2)]
            for t in range(N_DIM):
                v = plsc.load_gather(rows_ref, [r16, rot])
                xr = xrott[t, pl.ds(0, 16)]
                accs[t % 2] = accs[t % 2] + v * xr
                rot = (rot + 1) & (N_DIM - 1)
            acc = accs[0] + accs[1]
            logv[pl.ds(c * CHUNK + i * 16, 16)] = acc * INV_T
            return 0
        lax.fori_loop(0, CHUNK // 16, body, 0)

    def per_b(j, _):
        b = base + j
        pltpu.sync_copy(x_hbm.at[b], xv)
        pltpu.sync_copy(idx_hbm.at[b], idxv)
        for t in range(N_DIM):
            ridx = (iota + t) & (N_DIM - 1)
            xrott[t, pl.ds(0, 16)] = plsc.load_gather(xv, [ridx])
        pend = fire(0, rows_a, sem_a)
        for c in range(N_CHUNK):
            cur_rows = rows_a if c % 2 == 0 else rows_b
            if c < N_CHUNK - 1:
                nxt = fire(c + 1, rows_b if c % 2 == 0 else rows_a,
                           sem_b if c % 2 == 0 else sem_a)
            drain(pend)
            compute(c, cur_rows)
            if c < N_CHUNK - 1:
                pend = nxt
        pltpu.sync_copy(logv, lrest_hbm.at[b])
        return 0

    lax.fori_loop(0, B_PER_W, per_b, 0)


def _sc_scatter_body(y_hbm, mem_hbm, upd_hbm, newm_hbm, yv, trow,
                     cp_a, cp_b, sem_ra, sem_rb, sem_wa, sem_wb):
    wid = _wid()
    start = wid * ROWS_PER_W
    bufs = (cp_a, cp_b)
    rsems = (sem_ra, sem_rb)
    wsems = (sem_wa, sem_wb)
    rd = [None, None]
    wr = [None, None]

    def fire_read(c, p):
        rd[p] = pltpu.async_copy(
            mem_hbm.at[pl.ds(start + c * CP_CHUNK, CP_CHUNK)], bufs[p],
            rsems[p])

    def fire_write(c, p):
        wr[p] = pltpu.async_copy(
            bufs[p], newm_hbm.at[pl.ds(start + c * CP_CHUNK, CP_CHUNK)],
            wsems[p])

    fire_read(0, 0)
    for c in range(N_CP):
        p = c % 2
        if c + 1 < N_CP:
            if wr[1 - p] is not None:
                wr[1 - p].wait()
            fire_read(c + 1, 1 - p)
        rd[p].wait()
        fire_write(c, p)
    for p in range(2):
        if wr[p] is not None:
            wr[p].wait()

    pltpu.sync_copy(y_hbm, yv)

    def body(g, _):
        yvec = yv[pl.ds(g * 16, 16)]
        for l in range(16):
            yj = yvec[l]

            @pl.when(jnp.logical_and(yj >= start, yj < start + ROWS_PER_W))
            def _(yj=yj, l=l):
                pltpu.sync_copy(upd_hbm.at[pl.ds(g * 16 + l, 1)], trow)
                pltpu.sync_copy(trow, newm_hbm.at[pl.ds(yj, 1)])
        return 0

    lax.fori_loop(0, BSZ // 16, body, 0)


_MESH = plsc.VectorSubcoreMesh(core_axis_name="c", subcore_axis_name="s")

_sc_main = functools.partial(
    pl.kernel,
    mesh=_MESH,
    compiler_params=pltpu.CompilerParams(
        needs_layout_passes=False, use_tc_tiling_on_sc=False),
    out_type=[
        jax.ShapeDtypeStruct((BSZ, K), jnp.float32),
        jax.ShapeDtypeStruct((BSZ,), jnp.float32),
        jax.ShapeDtypeStruct((BSZ, N_DIM), jnp.float32),
    ],
    scratch_types=[
        pltpu.VMEM((N_DIM,), jnp.float32),
        pltpu.VMEM((K // 128, 128), jnp.int32),
        pltpu.VMEM((CHUNK, N_DIM), jnp.float32),
        pltpu.VMEM((CHUNK, N_DIM), jnp.float32),
        pltpu.VMEM((N_DIM, 16), jnp.float32),
        pltpu.VMEM((K,), jnp.float32),
        pltpu.VMEM((B_PER_W,), jnp.int32),
        pltpu.VMEM((B_PER_W, N_DIM), jnp.float32),
        pltpu.VMEM((B_PER_W, N_DIM), jnp.float32),
        pltpu.VMEM((B_PER_W, N_DIM), jnp.float32),
        pltpu.VMEM((B_PER_W,), jnp.float32),
        pltpu.SemaphoreType.DMA,
        pltpu.SemaphoreType.DMA,
        pltpu.SemaphoreType.DMA,
    ],
)(_sc_main_body)

_sc_scatter = functools.partial(
    pl.kernel,
    mesh=_MESH,
    compiler_params=pltpu.CompilerParams(
        needs_layout_passes=False, use_tc_tiling_on_sc=False),
    out_type=[jax.ShapeDtypeStruct((N_DATA, N_DIM), jnp.float32)],
    scratch_types=[
        pltpu.VMEM((BSZ,), jnp.int32),
        pltpu.VMEM((1, N_DIM), jnp.float32),
        pltpu.VMEM((CP_CHUNK, N_DIM), jnp.float32),
        pltpu.VMEM((CP_CHUNK, N_DIM), jnp.float32),
        pltpu.SemaphoreType.DMA,
        pltpu.SemaphoreType.DMA,
        pltpu.SemaphoreType.DMA,
        pltpu.SemaphoreType.DMA,
    ],
)(_sc_scatter_body)


def kernel(x, y, memory):
    idxc = _neg_idx()
    lrest, l0, upd = _sc_main(x, y, memory, idxc)
    (new_memory,) = _sc_scatter(y, memory, upd)
    logits = jnp.concatenate([l0[:, None], lrest], axis=1)
    labels = jnp.zeros((BSZ,), jnp.int32)
    return logits, labels, new_memory


# speedup vs baseline: 6.0325x; 1.0168x over previous
import functools

import jax
import jax.numpy as jnp
from jax import lax
from jax.experimental import pallas as pl
from jax.experimental.pallas import tpu as pltpu
from jax.experimental.pallas import tpu_sc as plsc

N_DATA = 1000000
N_DIM = 64
K = 2048
T = 0.07
M = 0.5
BSZ = 1024

NC = 2
NS = 16
NW = NC * NS
B_PER_W = BSZ // NW
ROWS_PER_W = N_DATA // NW
CHUNK = 512
N_CHUNK = K // CHUNK
CP_CHUNK = 625
N_CP = ROWS_PER_W // CP_CHUNK
INV_T = 1.0 / T

def _neg_idx():
    idx = jax.random.randint(
        jax.random.key(42), (BSZ, K + 1), 0, N_DATA, dtype=jnp.int32)
    return idx[:, 1:].reshape(BSZ, K // 128, 128)


def _rsqrt16(n):
    i = plsc.bitcast(n, jnp.int32)
    i = jnp.int32(0x5F3759DF) - (i >> 1)
    s = plsc.bitcast(i, jnp.float32)
    for _ in range(4):
        s = s * (1.5 - 0.5 * n * s * s)
    return s


def _wid():
    return lax.axis_index("s") * NC + lax.axis_index("c")


def _sc_main_body(x_hbm, y_hbm, mem_hbm, idx_hbm,
                  lrest_hbm, upd_hbm,
                  xv, idxv, rows_a, rows_b, xrott, logv,
                  yv, xtv, wposv, updv,
                  sem_a, sem_b, sem_u):
    wid = _wid()
    iota = lax.iota(jnp.int32, 16)
    iota64 = iota * N_DIM
    base = wid * B_PER_W

    pltpu.sync_copy(y_hbm.at[pl.ds(base, B_PER_W)], yv)
    pltpu.async_copy(mem_hbm.at[yv], wposv, sem_u).wait()
    pltpu.sync_copy(x_hbm.at[pl.ds(base, B_PER_W)], xtv)
    rinvs = []
    l0vecs = []
    for g in range(B_PER_W // 16):
        r16 = iota + g * 16
        accs = [jnp.zeros((16,), jnp.float32) for _ in range(4)]
        nrms = [jnp.zeros((16,), jnp.float32) for _ in range(4)]
        for d in range(N_DIM):
            col = jnp.full((16,), d, jnp.int32)
            wv = plsc.load_gather(wposv, [r16, col])
            xw = plsc.load_gather(xtv, [r16, col])
            accs[d % 4] = accs[d % 4] + wv * xw
            u = (wv + xw) * 0.5
            nrms[d % 4] = nrms[d % 4] + u * u
        dotv = (accs[0] + accs[1]) + (accs[2] + accs[3])
        nrm = (nrms[0] + nrms[1]) + (nrms[2] + nrms[3])
        l0vecs.append(dotv * INV_T)
        rinvs.append(_rsqrt16(jnp.maximum(nrm, 1e-24)))

    for j in range(B_PER_W):
        rs = rinvs[j // 16][j % 16] * 0.5
        for q in range(N_DIM // 16):
            sl = pl.ds(q * 16, 16)
            updv[j, sl] = (wposv[j, sl] + xtv[j, sl]) * rs

    pltpu.sync_copy(updv, upd_hbm.at[pl.ds(base, B_PER_W)])

    def fire(c, rows_ref, sem):
        ds_ = []
        for q in range(CHUNK // 128):
            ds_.append(pltpu.async_copy(
                mem_hbm.at[idxv.at[c * (CHUNK // 128) + q]],
                rows_ref.at[pl.ds(q * 128, 128)], sem))
        return ds_

    def drain(ds_):
        for d in ds_:
            d.wait()

    def compute(c, rows_ref, r):
        def body(i, _):
            r16 = iota + i * 16
            rot = iota
            accs = [jnp.zeros((16,), jnp.float32) for _ in range(2)]
            for t in range(N_DIM):
                v = plsc.load_gather(rows_ref, [r16, rot])
                xr = xrott[t, pl.ds(0, 16)]
                accs[t % 2] = accs[t % 2] + v * xr
                rot = (rot + 1) & (N_DIM - 1)
            acc = accs[0] + accs[1]
            plsc.store_scatter(
                logv,
                [jnp.zeros((16,), jnp.int32) + r,
                 iota + (c * CHUNK + i * 16 + 1)], acc * INV_T)
            return 0
        lax.fori_loop(0, CHUNK // 16, body, 0)

    def per_b(j, _):
        b = base + j
        pltpu.sync_copy(x_hbm.at[b], xv)
        pltpu.sync_copy(idx_hbm.at[b], idxv)
        r = j & 7
        l0sel = jnp.where(j < 16, l0vecs[0], l0vecs[1])
        plsc.store_scatter(logv,
                           [jnp.zeros((16,), jnp.int32) + r,
                            jnp.zeros((16,), jnp.int32)],
                           l0sel, mask=iota == (j & 15))
        for t in range(N_DIM):
            ridx = (iota + t) & (N_DIM - 1)
            xrott[t, pl.ds(0, 16)] = plsc.load_gather(xv, [ridx])
        pend = fire(0, rows_a, sem_a)
        for c in range(N_CHUNK):
            cur_rows = rows_a if c % 2 == 0 else rows_b
            if c < N_CHUNK - 1:
                nxt = fire(c + 1, rows_b if c % 2 == 0 else rows_a,
                           sem_b if c % 2 == 0 else sem_a)
            drain(pend)
            compute(c, cur_rows, r)
            if c < N_CHUNK - 1:
                pend = nxt
        @pl.when(r == 7)
        def _():
            pltpu.sync_copy(logv, lrest_hbm.at[pl.ds(b - 7, 8)])
        return 0

    lax.fori_loop(0, B_PER_W, per_b, 0)


def _sc_scatter_body(y_hbm, mem_hbm, upd_hbm, newm_hbm, yv, trow,
                     cp_a, cp_b, sem_ra, sem_rb, sem_wa, sem_wb):
    wid = _wid()
    start = wid * ROWS_PER_W
    bufs = (cp_a, cp_b)
    rsems = (sem_ra, sem_rb)
    wsems = (sem_wa, sem_wb)
    rd = [None, None]
    wr = [None, None]

    def fire_read(c, p):
        rd[p] = pltpu.async_copy(
            mem_hbm.at[pl.ds(start + c * CP_CHUNK, CP_CHUNK)], bufs[p],
            rsems[p])

    def fire_write(c, p):
        wr[p] = pltpu.async_copy(
            bufs[p], newm_hbm.at[pl.ds(start + c * CP_CHUNK, CP_CHUNK)],
            wsems[p])

    fire_read(0, 0)
    for c in range(N_CP):
        p = c % 2
        if c + 1 < N_CP:
            if wr[1 - p] is not None:
                wr[1 - p].wait()
            fire_read(c + 1, 1 - p)
        rd[p].wait()
        fire_write(c, p)
    for p in range(2):
        if wr[p] is not None:
            wr[p].wait()

    pltpu.sync_copy(y_hbm, yv)

    def body(g, _):
        yvec = yv[pl.ds(g * 16, 16)]
        for l in range(16):
            yj = yvec[l]

            @pl.when(jnp.logical_and(yj >= start, yj < start + ROWS_PER_W))
            def _(yj=yj, l=l):
                pltpu.sync_copy(upd_hbm.at[pl.ds(g * 16 + l, 1)], trow)
                pltpu.sync_copy(trow, newm_hbm.at[pl.ds(yj, 1)])
        return 0

    lax.fori_loop(0, BSZ // 16, body, 0)


_MESH = plsc.VectorSubcoreMesh(core_axis_name="c", subcore_axis_name="s")

_sc_main = functools.partial(
    pl.kernel,
    mesh=_MESH,
    compiler_params=pltpu.CompilerParams(
        needs_layout_passes=False, use_tc_tiling_on_sc=False),
    out_type=[
        jax.ShapeDtypeStruct((BSZ, K + 1), jnp.float32),
        jax.ShapeDtypeStruct((BSZ, N_DIM), jnp.float32),
    ],
    scratch_types=[
        pltpu.VMEM((N_DIM,), jnp.float32),
        pltpu.VMEM((K // 128, 128), jnp.int32),
        pltpu.VMEM((CHUNK, N_DIM), jnp.float32),
        pltpu.VMEM((CHUNK, N_DIM), jnp.float32),
        pltpu.VMEM((N_DIM, 16), jnp.float32),
        pltpu.VMEM((8, K + 1), jnp.float32),
        pltpu.VMEM((B_PER_W,), jnp.int32),
        pltpu.VMEM((B_PER_W, N_DIM), jnp.float32),
        pltpu.VMEM((B_PER_W, N_DIM), jnp.float32),
        pltpu.VMEM((B_PER_W, N_DIM), jnp.float32),
        pltpu.SemaphoreType.DMA,
        pltpu.SemaphoreType.DMA,
        pltpu.SemaphoreType.DMA,
    ],
)(_sc_main_body)

_sc_scatter = functools.partial(
    pl.kernel,
    mesh=_MESH,
    compiler_params=pltpu.CompilerParams(
        needs_layout_passes=False, use_tc_tiling_on_sc=False),
    out_type=[jax.ShapeDtypeStruct((N_DATA, N_DIM), jnp.float32)],
    scratch_types=[
        pltpu.VMEM((BSZ,), jnp.int32),
        pltpu.VMEM((1, N_DIM), jnp.float32),
        pltpu.VMEM((CP_CHUNK, N_DIM), jnp.float32),
        pltpu.VMEM((CP_CHUNK, N_DIM), jnp.float32),
        pltpu.SemaphoreType.DMA,
        pltpu.SemaphoreType.DMA,
        pltpu.SemaphoreType.DMA,
        pltpu.SemaphoreType.DMA,
    ],
)(_sc_scatter_body)


def kernel(x, y, memory):
    idxc = _neg_idx()
    logits, upd = _sc_main(x, y, memory, idxc)
    (new_memory,) = _sc_scatter(y, memory, upd)
    labels = jnp.zeros((BSZ,), jnp.int32)
    return logits, labels, new_memory


# speedup vs baseline: 6.2366x; 1.0338x over previous
import functools

import jax
import jax.numpy as jnp
from jax import lax
from jax.experimental import pallas as pl
from jax.experimental.pallas import tpu as pltpu
from jax.experimental.pallas import tpu_sc as plsc

N_DATA = 1000000
N_DIM = 64
K = 2048
T = 0.07
M = 0.5
BSZ = 1024

NC = 2
NS = 16
NW = NC * NS
B_PER_W = BSZ // NW
ROWS_PER_W = N_DATA // NW
CHUNK = 512
N_CHUNK = K // CHUNK
CP_FULL = ROWS_PER_W // CHUNK
CP_REM = ROWS_PER_W - CP_FULL * CHUNK
INV_T = 1.0 / T


def _neg_idx():
    idx = jax.random.randint(
        jax.random.key(42), (BSZ, K + 1), 0, N_DATA, dtype=jnp.int32)
    return idx[:, 1:].reshape(BSZ, K // 128, 128)


def _rsqrt16(n):
    i = plsc.bitcast(n, jnp.int32)
    i = jnp.int32(0x5F3759DF) - (i >> 1)
    s = plsc.bitcast(i, jnp.float32)
    for _ in range(4):
        s = s * (1.5 - 0.5 * n * s * s)
    return s


def _sc_body(x_hbm, y_hbm, mem_hbm, idx_hbm,
             logits_hbm, newm_hbm,
             xv, idxv, rows_a, rows_b, xrott, logv,
             yv, xtv, wposv, updv, yv_all, upd_sh,
             sem_a, sem_b, sem_u, sem_wa, sem_wb):
    cid = lax.axis_index("c")
    sid = lax.axis_index("s")
    wid = sid * NC + cid
    iota = lax.iota(jnp.int32, 16)

    l0vecs = []
    for half in range(2):
        hw = sid * NC + (cid if half == 0 else 1 - cid)
        hbase = hw * B_PER_W
        pltpu.sync_copy(y_hbm.at[pl.ds(hbase, B_PER_W)], yv)
        pltpu.async_copy(mem_hbm.at[yv], wposv, sem_u).wait()
        pltpu.sync_copy(x_hbm.at[pl.ds(hbase, B_PER_W)], xtv)

        rinvs = []
        for g in range(B_PER_W // 16):
            r16 = iota + g * 16
            accs = [jnp.zeros((16,), jnp.float32) for _ in range(4)]
            nrms = [jnp.zeros((16,), jnp.float32) for _ in range(4)]
            for d in range(N_DIM):
                col = jnp.full((16,), d, jnp.int32)
                wv = plsc.load_gather(wposv, [r16, col])
                xw = plsc.load_gather(xtv, [r16, col])
                accs[d % 4] = accs[d % 4] + wv * xw
                u = (wv + xw) * 0.5
                nrms[d % 4] = nrms[d % 4] + u * u
            nrm = (nrms[0] + nrms[1]) + (nrms[2] + nrms[3])
            rinvs.append(_rsqrt16(jnp.maximum(nrm, 1e-24)))
            if half == 0:
                dotv = (accs[0] + accs[1]) + (accs[2] + accs[3])
                l0vecs.append(dotv * INV_T)

        for j in range(B_PER_W):
            rs = rinvs[j // 16][j % 16] * 0.5
            for q in range(N_DIM // 16):
                sl = pl.ds(q * 16, 16)
                updv[j, sl] = (wposv[j, sl] + xtv[j, sl]) * rs

        pltpu.sync_copy(updv, upd_sh.at[pl.ds(hbase, B_PER_W)])

    plsc.subcore_barrier()

    start = wid * ROWS_PER_W
    bufs = (rows_a, rows_b)
    rsems = (sem_a, sem_b)
    wsems = (sem_wa, sem_wb)
    rd = [None, None]
    wr = [None, None]
    n_cp = CP_FULL + 1

    def cp_rows(c):
        return CHUNK if c < CP_FULL else CP_REM

    def fire_read(c, p):
        rd[p] = pltpu.async_copy(
            mem_hbm.at[pl.ds(start + c * CHUNK, cp_rows(c))],
            bufs[p].at[pl.ds(0, cp_rows(c))], rsems[p])

    def fire_write(c, p):
        wr[p] = pltpu.async_copy(
            bufs[p].at[pl.ds(0, cp_rows(c))],
            newm_hbm.at[pl.ds(start + c * CHUNK, cp_rows(c))], wsems[p])

    fire_read(0, 0)
    for c in range(n_cp):
        p = c % 2
        if c + 1 < n_cp:
            if wr[1 - p] is not None:
                wr[1 - p].wait()
            fire_read(c + 1, 1 - p)
        rd[p].wait()
        fire_write(c, p)
    for p in range(2):
        if wr[p] is not None:
            wr[p].wait()

    pltpu.sync_copy(y_hbm, yv_all)

    def patch(g, _):
        yvec = yv_all[pl.ds(g * 16, 16)]
        for l in range(16):
            yj = yvec[l]

            @pl.when(jnp.logical_and(yj >= start, yj < start + ROWS_PER_W))
            def _(yj=yj, l=l):
                pltpu.sync_copy(upd_sh.at[pl.ds(g * 16 + l, 1)],
                                newm_hbm.at[pl.ds(yj, 1)])
        return 0

    lax.fori_loop(0, BSZ // 16, patch, 0)

    base = wid * B_PER_W

    def fire(c, rows_ref, sem):
        ds_ = []
        for q in range(CHUNK // 128):
            ds_.append(pltpu.async_copy(
                mem_hbm.at[idxv.at[c * (CHUNK // 128) + q]],
                rows_ref.at[pl.ds(q * 128, 128)], sem))
        return ds_

    def drain(ds_):
        for d in ds_:
            d.wait()

    def compute(c, rows_ref, r):
        def body(i, _):
            r16 = iota + i * 16
            rot = iota
            accs = [jnp.zeros((16,), jnp.float32) for _ in range(2)]
            for t in range(N_DIM):
                v = plsc.load_gather(rows_ref, [r16, rot])
                xr = xrott[t, pl.ds(0, 16)]
                accs[t % 2] = accs[t % 2] + v * xr
                rot = (rot + 1) & (N_DIM - 1)
            acc = accs[0] + accs[1]
            plsc.store_scatter(
                logv,
                [jnp.zeros((16,), jnp.int32) + r,
                 iota + (c * CHUNK + i * 16 + 1)], acc * INV_T)
            return 0
        lax.fori_loop(0, CHUNK // 16, body, 0)

    def per_b(j, _):
        b = base + j
        pltpu.sync_copy(x_hbm.at[b], xv)
        pltpu.sync_copy(idx_hbm.at[b], idxv)
        r = j & 7
        l0sel = jnp.where(j < 16, l0vecs[0], l0vecs[1])
        plsc.store_scatter(logv,
                           [jnp.zeros((16,), jnp.int32) + r,
                            jnp.zeros((16,), jnp.int32)],
                           l0sel, mask=iota == (j & 15))
        for t in range(N_DIM):
            ridx = (iota + t) & (N_DIM - 1)
            xrott[t, pl.ds(0, 16)] = plsc.load_gather(xv, [ridx])
        pend = fire(0, rows_a, sem_a)
        for c in range(N_CHUNK):
            cur_rows = rows_a if c % 2 == 0 else rows_b
            if c < N_CHUNK - 1:
                nxt = fire(c + 1, rows_b if c % 2 == 0 else rows_a,
                           sem_b if c % 2 == 0 else sem_a)
            drain(pend)
            compute(c, cur_rows, r)
            if c < N_CHUNK - 1:
                pend = nxt

        @pl.when(r == 7)
        def _():
            pltpu.sync_copy(logv, logits_hbm.at[pl.ds(b - 7, 8)])
        return 0

    lax.fori_loop(0, B_PER_W, per_b, 0)


_MESH = plsc.VectorSubcoreMesh(core_axis_name="c", subcore_axis_name="s")

_sc_all = functools.partial(
    pl.kernel,
    mesh=_MESH,
    compiler_params=pltpu.CompilerParams(
        needs_layout_passes=False, use_tc_tiling_on_sc=False),
    out_type=[
        jax.ShapeDtypeStruct((BSZ, K + 1), jnp.float32),
        jax.ShapeDtypeStruct((N_DATA, N_DIM), jnp.float32),
    ],
    scratch_types=[
        pltpu.VMEM((N_DIM,), jnp.float32),
        pltpu.VMEM((K // 128, 128), jnp.int32),
        pltpu.VMEM((CHUNK, N_DIM), jnp.float32),
        pltpu.VMEM((CHUNK, N_DIM), jnp.float32),
        pltpu.VMEM((N_DIM, 16), jnp.float32),
        pltpu.VMEM((8, K + 1), jnp.float32),
        pltpu.VMEM((B_PER_W,), jnp.int32),
        pltpu.VMEM((B_PER_W, N_DIM), jnp.float32),
        pltpu.VMEM((B_PER_W, N_DIM), jnp.float32),
        pltpu.VMEM((B_PER_W, N_DIM), jnp.float32),
        pltpu.VMEM((BSZ,), jnp.int32),
        pltpu.VMEM_SHARED((BSZ, N_DIM), jnp.float32),
        pltpu.SemaphoreType.DMA,
        pltpu.SemaphoreType.DMA,
        pltpu.SemaphoreType.DMA,
        pltpu.SemaphoreType.DMA,
        pltpu.SemaphoreType.DMA,
    ],
)(_sc_body)


def kernel(x, y, memory):
    idxc = _neg_idx()
    logits, new_memory = _sc_all(x, y, memory, idxc)
    labels = jnp.zeros((BSZ,), jnp.int32)
    return logits, labels, new_memory
